# Initial kernel scaffold; baseline (speedup 1.0000x reference)
import jax
import jax.numpy as jnp
from jax.experimental import pallas as pl


def kernel(pred_logits, pred_boxes, target_sizes):
    raise NotImplementedError("write your pallas kernel here")


# speedup vs baseline: 5.5472x; 5.5472x over previous
import functools

import jax
import jax.numpy as jnp
from jax import lax
from jax.experimental import pallas as pl

_K = 100
_KP = 128


def _peel_body(logits_ref, boxes_ref, ts_ref,
               scores_ref, labels_ref, x0_ref, y0_ref, x1_ref, y1_ref,
               *, Q, C):
    iota_q = lax.broadcasted_iota(jnp.int32, (1, Q), 1)
    iota_c = lax.broadcasted_iota(jnp.int32, (1, C), 1)
    iota_k = lax.broadcasted_iota(jnp.int32, (1, _KP), 1)

    rm = jnp.max(logits_ref[...], axis=-1)

    zf = jnp.zeros((1, _KP), jnp.float32)
    zi = jnp.zeros((1, _KP), jnp.int32)

    def body(k, carry):
        rm, sc, lb, cx, cy, w, h = carry
        m = jnp.max(rm)
        q = jnp.min(jnp.where(rm == m, iota_q, Q))
        row = logits_ref[0, pl.ds(q, 1), :]
        c = jnp.min(jnp.where(row == m, iota_c, C))
        hit = iota_k == k
        sc = jnp.where(hit, m, sc)
        lb = jnp.where(hit, c, lb)
        b4 = boxes_ref[0, pl.ds(q, 1), :]
        cx = jnp.where(hit, b4[:, 0:1], cx)
        cy = jnp.where(hit, b4[:, 1:2], cy)
        w = jnp.where(hit, b4[:, 2:3], w)
        h = jnp.where(hit, b4[:, 3:4], h)
        newrow = jnp.where(iota_c == c, -jnp.inf, row)
        logits_ref[0, pl.ds(q, 1), :] = newrow
        rm = jnp.where(iota_q == q, jnp.max(newrow), rm)
        return rm, sc, lb, cx, cy, w, h

    _, sc, lb, cx, cy, w, h = lax.fori_loop(
        0, _K, body, (rm, zf, zi, zf, zf, zf, zf))

    scores_ref[0] = 1.0 / (1.0 + jnp.exp(-sc))
    labels_ref[0] = lb
    img_h = ts_ref[0, 0, 0].astype(jnp.float32)
    img_w = ts_ref[0, 0, 1].astype(jnp.float32)
    x0_ref[0] = (cx - 0.5 * w) * img_w
    y0_ref[0] = (cy - 0.5 * h) * img_h
    x1_ref[0] = (cx + 0.5 * w) * img_w
    y1_ref[0] = (cy + 0.5 * h) * img_h


def kernel(pred_logits, pred_boxes, target_sizes):
    B, Q, C = pred_logits.shape
    ts3 = target_sizes.reshape(B, 1, 2).astype(jnp.int32)
    f32 = jnp.float32
    outs = pl.pallas_call(
        functools.partial(_peel_body, Q=Q, C=C),
        grid=(B,),
        in_specs=[
            pl.BlockSpec((1, Q, C), lambda b: (b, 0, 0)),
            pl.BlockSpec((1, Q, 4), lambda b: (b, 0, 0)),
            pl.BlockSpec((1, 1, 2), lambda b: (b, 0, 0)),
        ],
        out_specs=[pl.BlockSpec((1, 1, _KP), lambda b: (b, 0, 0))] * 6,
        out_shape=[
            jax.ShapeDtypeStruct((B, 1, _KP), f32),
            jax.ShapeDtypeStruct((B, 1, _KP), jnp.int32),
            jax.ShapeDtypeStruct((B, 1, _KP), f32),
            jax.ShapeDtypeStruct((B, 1, _KP), f32),
            jax.ShapeDtypeStruct((B, 1, _KP), f32),
            jax.ShapeDtypeStruct((B, 1, _KP), f32),
        ],
    )(pred_logits, pred_boxes, ts3)
    scores, labels, x0, y0, x1, y1 = (o[:, 0, :] for o in outs)
    boxes = jnp.stack([x0, y0, x1, y1], axis=-1)[:, :_K, :]
    return scores[:, :_K], labels[:, :_K], boxes


# speedup vs baseline: 6.3412x; 1.1431x over previous
import functools

import jax
import jax.numpy as jnp
from jax import lax
from jax.experimental import pallas as pl
from jax.experimental.pallas import tpu as pltpu
from jax.experimental.pallas import tpu_sc as plsc

_L = 16
_K = 100
_KP = 128
_CH = 1024
_WIN = 8192


def _neg():
    return jnp.full((_L,), -jnp.inf, jnp.float32)


def _sstore(ref, idx, val_vec):
    lane = lax.broadcasted_iota(jnp.int32, (_L,), 0)
    plsc.store_scatter(ref, [jnp.full((_L,), idx, jnp.int32)],
                       val_vec, mask=lane == 0)


def _perm(v, s):
    lane = lax.broadcasted_iota(jnp.int32, (_L,), 0)
    return v.at[lane ^ s].get(mode="promise_in_bounds")


def _pmax(v):
    for s in (8, 4, 2, 1):
        v = jnp.maximum(v, _perm(v, s))
    return v


def _pmin(v):
    for s in (8, 4, 2, 1):
        v = jnp.minimum(v, _perm(v, s))
    return v


@functools.lru_cache(maxsize=None)
def _make_sc(B, Q, C):
    N = Q * C
    NCH = -(-(N + 4) // _CH)
    NCHP = -(-NCH // _L) * _L
    NWF = (NCH - 2) // 8
    TAIL_OFF = NWF * _WIN
    LASTN = N + 4 - (NCH - 1) * _CH
    LASTV = -(-LASTN // _L)
    mesh = plsc.VectorSubcoreMesh(
        core_axis_name="c", subcore_axis_name="s",
        num_cores=2, num_subcores=16)
    lane = lambda: lax.broadcasted_iota(jnp.int32, (_L,), 0)
    NEGINF = jnp.float32(-jnp.inf)

    @functools.partial(
        pl.kernel,
        out_type=[
            jax.ShapeDtypeStruct((B * _KP,), jnp.float32),
            jax.ShapeDtypeStruct((B * _KP,), jnp.int32),
            jax.ShapeDtypeStruct((B * _KP * 4,), jnp.float32),
        ],
        mesh=mesh,
        compiler_params=pltpu.CompilerParams(needs_layout_passes=False),
        scratch_types=[
            pltpu.VMEM((_WIN,), jnp.float32),
            pltpu.VMEM((_WIN,), jnp.float32),
            pltpu.VMEM((_CH,), jnp.float32),
            pltpu.VMEM((NCHP,), jnp.float32),
            pltpu.VMEM((Q * 4,), jnp.float32),
            pltpu.VMEM((2 * B + _L,), jnp.int32),
            pltpu.VMEM((_KP,), jnp.float32),
            pltpu.VMEM((_KP,), jnp.int32),
            pltpu.VMEM((_KP * 4,), jnp.float32),
            pltpu.SemaphoreType.DMA((2,)),
        ],
    )
    def sc_kernel(lg, bx, ts, sc_out, lb_out, bo_out,
                  wb0, wb1, cbuf, cmax, rowbox, tsv, vval, vidx, obox, sems):
        b = lax.axis_index("s") * 2 + lax.axis_index("c")
        rstart = b * N
        astart = (rstart // 8) * 8
        lead = rstart - astart
        rlen = lead + N

        pltpu.sync_copy(bx.at[pl.ds(b * Q * 4, Q * 4)], rowbox)
        pltpu.sync_copy(ts, tsv.at[pl.ds(0, 2 * B)])

        pltpu.async_copy(lg.at[pl.ds(astart, _WIN)], wb0, sems.at[0])

        def chunk_maxes(buf, w):
            for k8 in range(8):
                macc = _neg()
                for i in range(_CH // _L):
                    macc = jnp.maximum(
                        macc, buf[pl.ds((k8 * (_CH // _L) + i) * _L, _L)])
                _sstore(cmax, w * 8 + k8, _pmax(macc))

        def w_body(wp, _):
            w0 = 2 * wp
            pltpu.make_async_copy(
                lg.at[pl.ds(astart + w0 * _WIN, _WIN)], wb0, sems.at[0]).wait()
            pltpu.async_copy(
                lg.at[pl.ds(astart + (w0 + 1) * _WIN, _WIN)], wb1, sems.at[1])

            @pl.when(wp == 0)
            def _():
                v0 = wb0[pl.ds(0, _L)]
                wb0[pl.ds(0, _L)] = jnp.where(lane() < lead, NEGINF, v0)

            chunk_maxes(wb0, w0)
            pltpu.make_async_copy(
                lg.at[pl.ds(astart + (w0 + 1) * _WIN, _WIN)], wb1,
                sems.at[1]).wait()

            @pl.when(w0 + 2 < NWF)
            def _():
                pltpu.async_copy(
                    lg.at[pl.ds(astart + (w0 + 2) * _WIN, _WIN)], wb0,
                    sems.at[0])

            chunk_maxes(wb1, w0 + 1)
            return 0

        lax.fori_loop(0, NWF // 2, w_body, 0)

        pltpu.sync_copy(
            lg.at[pl.ds(astart + TAIL_OFF, _CH + LASTN)],
            wb0.at[pl.ds(0, _CH + LASTN)])
        cmax[pl.ds(NCHP - _L, _L)] = _neg()
        macc = _neg()
        for i in range(_CH // _L):
            macc = jnp.maximum(macc, wb0[pl.ds(i * _L, _L)])
        _sstore(cmax, NCH - 2, _pmax(macc))
        vlen = rlen - (NCH - 1) * _CH
        macc = _neg()
        for i in range(LASTV):
            v = wb0[pl.ds(_CH + i * _L, _L)]
            macc = jnp.maximum(macc, jnp.where(i * _L + lane() < vlen, v, NEGINF))
        _sstore(cmax, NCH - 1, _pmax(macc))

        def peel(k, _):
            macc = _neg()
            iacc = jnp.zeros((_L,), jnp.int32)
            for i in range(NCHP // _L):
                v = cmax[pl.ds(i * _L, _L)]
                better = v > macc
                macc = jnp.where(better, v, macc)
                iacc = jnp.where(better, jnp.int32(i), iacc)
            m = _pmax(macc)
            jbest = _pmin(
                jnp.where(macc == m, iacc * _L + lane(), jnp.int32(1 << 30)))[0]
            cstart = astart + jbest * _CH

            @pl.when(jbest < NCH - 1)
            def _():
                pltpu.sync_copy(lg.at[pl.ds(cstart, _CH)], cbuf)

            @pl.when(jbest == NCH - 1)
            def _():
                for i in range(_CH // _L):
                    cbuf[pl.ds(i * _L, _L)] = _neg()
                pltpu.sync_copy(lg.at[pl.ds(cstart, LASTN)],
                                cbuf.at[pl.ds(0, LASTN)])
                vl = rlen - (NCH - 1) * _CH
                for i in range(LASTV):
                    v = cbuf[pl.ds(i * _L, _L)]
                    cbuf[pl.ds(i * _L, _L)] = jnp.where(
                        i * _L + lane() < vl, v, NEGINF)

            @pl.when(jbest == 0)
            def _():
                v0 = cbuf[pl.ds(0, _L)]
                cbuf[pl.ds(0, _L)] = jnp.where(lane() < lead, NEGINF, v0)

            def ex_body(j, _):
                rel = vidx[pl.ds(j, _L)][0] + lead - jbest * _CH

                @pl.when((rel >= 0) & (rel < _CH))
                def _():
                    _sstore(cbuf, rel, _neg())
                return 0

            lax.fori_loop(0, k, ex_body, 0)

            emacc = _neg()
            eiacc = jnp.zeros((_L,), jnp.int32)
            for i in range(_CH // _L):
                v = cbuf[pl.ds(i * _L, _L)]
                better = v > emacc
                emacc = jnp.where(better, v, emacc)
                eiacc = jnp.where(better, jnp.int32(i), eiacc)
            me = _pmax(emacc)
            pos = _pmin(
                jnp.where(emacc == me, eiacc * _L + lane(),
                          jnp.int32(1 << 30)))[0]
            _sstore(vval, k, me)
            _sstore(vidx, k,
                    jnp.full((_L,), jbest * _CH + pos - lead, jnp.int32))
            _sstore(cbuf, pos, _neg())
            macc2 = _neg()
            for i in range(_CH // _L):
                macc2 = jnp.maximum(macc2, cbuf[pl.ds(i * _L, _L)])
            _sstore(cmax, jbest, _pmax(macc2))
            return 0

        lax.fori_loop(0, _K, peel, 0)
        pad = lane() >= _K - 96
        tv = vval[pl.ds(96, _L)]
        vval[pl.ds(96, _L)] = jnp.where(pad, jnp.float32(0.0), tv)
        ti = vidx[pl.ds(96, _L)]
        vidx[pl.ds(96, _L)] = jnp.where(pad, jnp.int32(0), ti)
        vval[pl.ds(112, _L)] = jnp.zeros((_L,), jnp.float32)
        vidx[pl.ds(112, _L)] = jnp.zeros((_L,), jnp.int32)

        tpair = tsv[pl.ds(2 * b, _L)]
        img_h = tpair[0].astype(jnp.float32)
        img_w = tpair[1].astype(jnp.float32)
        for g in range(_KP // _L):
            v = vval[pl.ds(g * _L, _L)]
            li = vidx[pl.ds(g * _L, _L)]
            qr = li // C
            vval[pl.ds(g * _L, _L)] = 1.0 / (1.0 + jnp.exp(-v))
            vidx[pl.ds(g * _L, _L)] = li - qr * C
            r4 = qr * 4
            cx = plsc.load_gather(rowbox, [r4])
            cy = plsc.load_gather(rowbox, [r4 + 1])
            ww = plsc.load_gather(rowbox, [r4 + 2])
            hh = plsc.load_gather(rowbox, [r4 + 3])
            j4 = (g * _L + lane()) * 4
            plsc.store_scatter(obox, [j4], (cx - 0.5 * ww) * img_w)
            plsc.store_scatter(obox, [j4 + 1], (cy - 0.5 * hh) * img_h)
            plsc.store_scatter(obox, [j4 + 2], (cx + 0.5 * ww) * img_w)
            plsc.store_scatter(obox, [j4 + 3], (cy + 0.5 * hh) * img_h)

        pltpu.sync_copy(vval, sc_out.at[pl.ds(b * _KP, _KP)])
        pltpu.sync_copy(vidx, lb_out.at[pl.ds(b * _KP, _KP)])
        pltpu.sync_copy(obox, bo_out.at[pl.ds(b * _KP * 4, _KP * 4)])

    return sc_kernel


def kernel(pred_logits, pred_boxes, target_sizes):
    B, Q, C = pred_logits.shape
    sc = _make_sc(B, Q, C)
    scores, labels, boxes = sc(
        pred_logits.reshape(-1),
        pred_boxes.reshape(-1),
        target_sizes.astype(jnp.int32).reshape(-1),
    )
    return (scores.reshape(B, _KP)[:, :_K],
            labels.reshape(B, _KP)[:, :_K],
            boxes.reshape(B, _KP, 4)[:, :_K, :])


# speedup vs baseline: 6.6396x; 1.0470x over previous
import functools

import jax
import jax.numpy as jnp
from jax import lax
from jax.experimental import pallas as pl
from jax.experimental.pallas import tpu as pltpu
from jax.experimental.pallas import tpu_sc as plsc

_L = 16
_K = 100
_KP = 128
_CH = 1024
_WIN = 8192


def _neg():
    return jnp.full((_L,), -jnp.inf, jnp.float32)


def _sstore(ref, idx, val_vec):
    lane = lax.broadcasted_iota(jnp.int32, (_L,), 0)
    plsc.store_scatter(ref, [jnp.full((_L,), idx, jnp.int32)],
                       val_vec, mask=lane == 0)


def _perm(v, s):
    lane = lax.broadcasted_iota(jnp.int32, (_L,), 0)
    return v.at[lane ^ s].get(mode="promise_in_bounds")


def _pmax(v):
    for s in (8, 4, 2, 1):
        v = jnp.maximum(v, _perm(v, s))
    return v


def _pmin(v):
    for s in (8, 4, 2, 1):
        v = jnp.minimum(v, _perm(v, s))
    return v


@functools.lru_cache(maxsize=None)
def _make_sc(B, Q, C):
    N = Q * C
    TOT = B * N
    NCH = -(-(N + 12) // _CH)
    NCHP = -(-NCH // _L) * _L
    NWF = (NCH - 2) // 8
    TAIL_OFF = NWF * _WIN
    LASTN = N + 12 - (NCH - 1) * _CH
    LASTNS = ((N + 4 - (NCH - 1) * _CH) // 8) * 8
    LASTV = -(-LASTN // _L)
    mesh = plsc.VectorSubcoreMesh(
        core_axis_name="c", subcore_axis_name="s",
        num_cores=2, num_subcores=16)
    lane = lambda: lax.broadcasted_iota(jnp.int32, (_L,), 0)
    NEGINF = jnp.float32(-jnp.inf)

    @functools.partial(
        pl.kernel,
        out_type=[
            jax.ShapeDtypeStruct((B * _KP,), jnp.float32),
            jax.ShapeDtypeStruct((B * _KP,), jnp.int32),
            jax.ShapeDtypeStruct((B * _KP * 4,), jnp.float32),
        ],
        mesh=mesh,
        compiler_params=pltpu.CompilerParams(needs_layout_passes=False),
        scratch_types=[
            pltpu.VMEM((_WIN,), jnp.float32),
            pltpu.VMEM((_WIN,), jnp.float32),
            pltpu.VMEM((_CH,), jnp.float32),
            pltpu.VMEM((NCHP,), jnp.float32),
            pltpu.VMEM((Q * 4,), jnp.float32),
            pltpu.VMEM((2 * B + _L,), jnp.int32),
            pltpu.VMEM((_KP,), jnp.float32),
            pltpu.VMEM((_KP,), jnp.int32),
            pltpu.VMEM((_KP * 4,), jnp.float32),
            pltpu.SemaphoreType.DMA((2,)),
        ],
    )
    def sc_kernel(lg, bx, ts, sc_out, lb_out, bo_out,
                  wb0, wb1, cbuf, cmax, rowbox, tsv, vval, vidx, obox, sems):
        b = lax.axis_index("s") * 2 + lax.axis_index("c")
        rstart = b * N
        astart = (rstart // 16) * 16
        lead = rstart - astart
        rlen = lead + N
        tail_base = astart + TAIL_OFF + _CH
        long_tail = tail_base + LASTN <= TOT

        pltpu.sync_copy(bx.at[pl.ds(b * Q * 4, Q * 4)], rowbox)
        pltpu.sync_copy(ts, tsv.at[pl.ds(0, 2 * B)])
        for g in range(_KP // _L):
            vidx[pl.ds(g * _L, _L)] = jnp.full((_L,), -1, jnp.int32)

        pltpu.async_copy(lg.at[pl.ds(astart, _WIN)], wb0, sems.at[0])

        def chunk_maxes(buf, w):
            for k8 in range(8):
                macc = _neg()
                for i in range(_CH // _L):
                    macc = jnp.maximum(
                        macc, buf[pl.ds((k8 * (_CH // _L) + i) * _L, _L)])
                _sstore(cmax, w * 8 + k8, _pmax(macc))

        def w_body(wp, _):
            w0 = 2 * wp
            pltpu.make_async_copy(
                lg.at[pl.ds(astart + w0 * _WIN, _WIN)], wb0, sems.at[0]).wait()
            pltpu.async_copy(
                lg.at[pl.ds(astart + (w0 + 1) * _WIN, _WIN)], wb1, sems.at[1])

            @pl.when(wp == 0)
            def _():
                v0 = wb0[pl.ds(0, _L)]
                wb0[pl.ds(0, _L)] = jnp.where(lane() < lead, NEGINF, v0)

            chunk_maxes(wb0, w0)
            pltpu.make_async_copy(
                lg.at[pl.ds(astart + (w0 + 1) * _WIN, _WIN)], wb1,
                sems.at[1]).wait()

            @pl.when(w0 + 2 < NWF)
            def _():
                pltpu.async_copy(
                    lg.at[pl.ds(astart + (w0 + 2) * _WIN, _WIN)], wb0,
                    sems.at[0])

            chunk_maxes(wb1, w0 + 1)
            return 0

        lax.fori_loop(0, NWF // 2, w_body, 0)

        pltpu.sync_copy(
            lg.at[pl.ds(astart + TAIL_OFF, _CH)], wb0.at[pl.ds(0, _CH)])

        @pl.when(long_tail)
        def _():
            pltpu.sync_copy(lg.at[pl.ds(tail_base, LASTN)],
                            wb0.at[pl.ds(_CH, LASTN)])

        @pl.when(jnp.logical_not(long_tail))
        def _():
            pltpu.sync_copy(lg.at[pl.ds(tail_base, LASTNS)],
                            wb0.at[pl.ds(_CH, LASTNS)])
        cmax[pl.ds(NCHP - _L, _L)] = _neg()
        macc = _neg()
        for i in range(_CH // _L):
            macc = jnp.maximum(macc, wb0[pl.ds(i * _L, _L)])
        _sstore(cmax, NCH - 2, _pmax(macc))
        vlen = rlen - (NCH - 1) * _CH
        macc = _neg()
        for i in range(LASTV):
            v = wb0[pl.ds(_CH + i * _L, _L)]
            macc = jnp.maximum(macc, jnp.where(i * _L + lane() < vlen, v, NEGINF))
        _sstore(cmax, NCH - 1, _pmax(macc))

        def peel(k, _):
            macc = _neg()
            iacc = jnp.zeros((_L,), jnp.int32)
            for i in range(NCHP // _L):
                v = cmax[pl.ds(i * _L, _L)]
                better = v > macc
                macc = jnp.where(better, v, macc)
                iacc = jnp.where(better, jnp.int32(i), iacc)
            m = _pmax(macc)
            jbest = _pmin(
                jnp.where(macc == m, iacc * _L + lane(), jnp.int32(1 << 30)))[0]
            cstart = astart + jbest * _CH

            @pl.when(jbest < NCH - 1)
            def _():
                pltpu.sync_copy(lg.at[pl.ds(cstart, _CH)], cbuf)

            @pl.when(jbest == NCH - 1)
            def _():
                for i in range(_CH // _L):
                    cbuf[pl.ds(i * _L, _L)] = _neg()

                @pl.when(long_tail)
                def _():
                    pltpu.sync_copy(lg.at[pl.ds(cstart, LASTN)],
                                    cbuf.at[pl.ds(0, LASTN)])

                @pl.when(jnp.logical_not(long_tail))
                def _():
                    pltpu.sync_copy(lg.at[pl.ds(cstart, LASTNS)],
                                    cbuf.at[pl.ds(0, LASTNS)])

                vl = rlen - (NCH - 1) * _CH
                for i in range(LASTV):
                    v = cbuf[pl.ds(i * _L, _L)]
                    cbuf[pl.ds(i * _L, _L)] = jnp.where(
                        i * _L + lane() < vl, v, NEGINF)

            @pl.when(jbest == 0)
            def _():
                v0 = cbuf[pl.ds(0, _L)]
                cbuf[pl.ds(0, _L)] = jnp.where(lane() < lead, NEGINF, v0)

            for g in range(_KP // _L):
                rel = vidx[pl.ds(g * _L, _L)] + lead - jbest * _CH
                inside = (rel >= 0) & (rel < _CH)
                relc = jnp.where(inside, rel, 0)
                plsc.store_scatter(cbuf, [relc], _neg(), mask=inside)

            emacc = _neg()
            eiacc = jnp.zeros((_L,), jnp.int32)
            for i in range(_CH // _L):
                v = cbuf[pl.ds(i * _L, _L)]
                better = v > emacc
                emacc = jnp.where(better, v, emacc)
                eiacc = jnp.where(better, jnp.int32(i), eiacc)
            me = _pmax(emacc)
            pos = _pmin(
                jnp.where(emacc == me, eiacc * _L + lane(),
                          jnp.int32(1 << 30)))[0]
            _sstore(vval, k, me)
            _sstore(vidx, k,
                    jnp.full((_L,), jbest * _CH + pos - lead, jnp.int32))
            _sstore(cbuf, pos, _neg())
            macc2 = _neg()
            for i in range(_CH // _L):
                macc2 = jnp.maximum(macc2, cbuf[pl.ds(i * _L, _L)])
            _sstore(cmax, jbest, _pmax(macc2))
            return 0

        lax.fori_loop(0, _K, peel, 0)
        pad = lane() >= _K - 96
        tv = vval[pl.ds(96, _L)]
        vval[pl.ds(96, _L)] = jnp.where(pad, jnp.float32(0.0), tv)
        ti = vidx[pl.ds(96, _L)]
        vidx[pl.ds(96, _L)] = jnp.where(pad, jnp.int32(0), ti)
        vval[pl.ds(112, _L)] = jnp.zeros((_L,), jnp.float32)
        vidx[pl.ds(112, _L)] = jnp.zeros((_L,), jnp.int32)

        tpair = tsv[pl.ds(2 * b, _L)]
        img_h = tpair[0].astype(jnp.float32)
        img_w = tpair[1].astype(jnp.float32)
        for g in range(_KP // _L):
            v = vval[pl.ds(g * _L, _L)]
            li = vidx[pl.ds(g * _L, _L)]
            qr = li // C
            vval[pl.ds(g * _L, _L)] = 1.0 / (1.0 + jnp.exp(-v))
            vidx[pl.ds(g * _L, _L)] = li - qr * C
            r4 = qr * 4
            cx = plsc.load_gather(rowbox, [r4])
            cy = plsc.load_gather(rowbox, [r4 + 1])
            ww = plsc.load_gather(rowbox, [r4 + 2])
            hh = plsc.load_gather(rowbox, [r4 + 3])
            j4 = (g * _L + lane()) * 4
            plsc.store_scatter(obox, [j4], (cx - 0.5 * ww) * img_w)
            plsc.store_scatter(obox, [j4 + 1], (cy - 0.5 * hh) * img_h)
            plsc.store_scatter(obox, [j4 + 2], (cx + 0.5 * ww) * img_w)
            plsc.store_scatter(obox, [j4 + 3], (cy + 0.5 * hh) * img_h)

        pltpu.sync_copy(vval, sc_out.at[pl.ds(b * _KP, _KP)])
        pltpu.sync_copy(vidx, lb_out.at[pl.ds(b * _KP, _KP)])
        pltpu.sync_copy(obox, bo_out.at[pl.ds(b * _KP * 4, _KP * 4)])

    return sc_kernel


def kernel(pred_logits, pred_boxes, target_sizes):
    B, Q, C = pred_logits.shape
    sc = _make_sc(B, Q, C)
    scores, labels, boxes = sc(
        pred_logits.reshape(-1),
        pred_boxes.reshape(-1),
        target_sizes.astype(jnp.int32).reshape(-1),
    )
    return (scores.reshape(B, _KP)[:, :_K],
            labels.reshape(B, _KP)[:, :_K],
            boxes.reshape(B, _KP, 4)[:, :_K, :])
